# baseline (speedup 1.0000x reference)
import functools
import jax
import jax.numpy as jnp
from jax import lax
from jax.experimental import pallas as pl
from jax.experimental.pallas import tpu as pltpu
from jax.experimental.pallas import tpu_sc as plsc

_E, _B, _D1, _D2 = 32, 32, 256, 256
_NW = 32
_RW = _D1 // _NW
_RC = 4
_NCH = _RW // _RC
_L = 16
_CV = _D2 // _L

_mesh = plsc.VectorSubcoreMesh(core_axis_name="c", subcore_axis_name="s")


def _sc_body(wT_hbm, W_hbm, out_hbm, wT_v, slab_a, slab_b, out_v, sem_a, sem_b):
    wid = lax.axis_index("s") * 2 + lax.axis_index("c")
    row0 = wid * _RW
    pltpu.sync_copy(wT_hbm, wT_v)

    slabs = (slab_a, slab_b)
    sems = (sem_a, sem_b)
    copies = []
    for ci in range(_NCH):
        copies.append(pltpu.async_copy(
            W_hbm.at[:, pl.ds(row0 + ci * _RC, _RC), :], slabs[ci], sems[ci]))

    for ci in range(_NCH):
        slab = slabs[ci]
        copies[ci].wait()

        def b_body(b4, _, slab=slab):
            bs = [b4 * 4 + i for i in range(4)]
            svals = []
            for b in bs:
                w0 = wT_v[b, pl.ds(0, _L)]
                w1 = wT_v[b, pl.ds(_L, _L)]
                svals.append([w0[i] for i in range(_L)]
                             + [w1[i] for i in range(_L)])

            def c_body(c, _):
                co = c * _L
                acc = [[None] * _RC for _ in range(4)]
                for r in range(_RC):
                    x = slab[0, r, pl.ds(co, _L)]
                    for k in range(4):
                        acc[k][r] = svals[k][0] * x
                for e in range(1, _E):
                    for r in range(_RC):
                        x = slab[e, r, pl.ds(co, _L)]
                        for k in range(4):
                            acc[k][r] = acc[k][r] + svals[k][e] * x
                for k in range(4):
                    for r in range(_RC):
                        out_v[bs[k], r, pl.ds(co, _L)] = acc[k][r]
                return 0

            lax.fori_loop(0, _CV, c_body, 0)
            return 0

        lax.fori_loop(0, _B // 4, b_body, 0)
        pltpu.sync_copy(out_v, out_hbm.at[:, pl.ds(row0 + ci * _RC, _RC), :])


def kernel(superposition_weights, W):
    k = pl.kernel(
        _sc_body,
        out_type=jax.ShapeDtypeStruct((_B, _D1, _D2), jnp.float32),
        mesh=_mesh,
        scratch_types=[
            pltpu.VMEM((_B, _E), jnp.float32),
            pltpu.VMEM((_E, _RC, _D2), jnp.float32),
            pltpu.VMEM((_E, _RC, _D2), jnp.float32),
            pltpu.VMEM((_B, _RC, _D2), jnp.float32),
            pltpu.SemaphoreType.DMA,
            pltpu.SemaphoreType.DMA,
        ],
    )
    return k(superposition_weights.T, W)
